# baseline (speedup 1.0000x reference)
import functools

import jax
import jax.numpy as jnp
from jax import lax
from jax.experimental import pallas as pl
from jax.experimental.pallas import tpu as pltpu
from jax.experimental.pallas import tpu_sc as plsc


def _sc_transpose(table_t):
    d, v = table_t.shape
    full_cols = v // 128
    tail = v - full_cols * 128
    info = plsc.get_sparse_core_info()
    nw = info.num_cores * info.num_subcores
    steps = (full_cols + nw - 1) // nw

    mesh = plsc.VectorSubcoreMesh(core_axis_name="c", subcore_axis_name="s")

    @functools.partial(
        pl.kernel,
        mesh=mesh,
        compiler_params=pltpu.CompilerParams(use_tc_tiling_on_sc=True,
                                             needs_layout_passes=False),
        out_type=jax.ShapeDtypeStruct((v * d,), jnp.float32),
        scratch_types=[
            pltpu.VMEM((d, 128), jnp.float32),
            pltpu.VMEM((d, tail), jnp.float32) if tail else None,
            pltpu.VMEM((128 * d,), jnp.float32),
        ],
    )
    def transpose_k(tt_hbm, out_hbm, colbuf, tailbuf, rowbuf):
        wid = lax.axis_index("s") * info.num_cores + lax.axis_index("c")
        d0 = jnp.arange(16, dtype=jnp.int32)

        def do_col(c):
            pltpu.sync_copy(tt_hbm.at[:, pl.ds(c * 128, 128)], colbuf)

            def row_body(i, _):
                iv = jnp.full((16,), i, dtype=jnp.int32)
                for h in range(d // 16):
                    vals = plsc.load_gather(colbuf, [d0 + h * 16, iv])
                    rowbuf[pl.ds(i * d + h * 16, 16)] = vals
                return 0

            lax.fori_loop(0, 128, row_body, 0)
            pltpu.sync_copy(rowbuf, out_hbm.at[pl.ds(c * 128 * d, 128 * d)])

        def step_body(t, _):
            c = wid + t * nw

            @pl.when(c < full_cols)
            def _():
                do_col(c)

            return 0

        lax.fori_loop(0, steps, step_body, 0)

        if tail:
            @pl.when(wid == 0)
            def _():
                pltpu.sync_copy(tt_hbm.at[:, pl.ds(full_cols * 128, tail)],
                                tailbuf)

                def trow_body(i, _):
                    iv = jnp.full((16,), i, dtype=jnp.int32)
                    for h in range(d // 16):
                        vals = plsc.load_gather(tailbuf, [d0 + h * 16, iv])
                        rowbuf[pl.ds(i * d + h * 16, 16)] = vals
                    return 0

                lax.fori_loop(0, tail, trow_body, 0)
                pltpu.sync_copy(rowbuf.at[pl.ds(0, tail * d)],
                                out_hbm.at[pl.ds(full_cols * 128 * d, tail * d)])

    return transpose_k(table_t)


def _sc_gather(table, seq_idx_t, item_idx):
    info = plsc.get_sparse_core_info()
    nw = info.num_cores * info.num_subcores
    n_seq = seq_idx_t.shape[0]
    n_tgt = item_idx.shape[0]
    d = table.shape[1]
    seq_pw = n_seq // nw
    tgt_pw = n_tgt // nw
    ch = 800
    n_ch = seq_pw // ch

    mesh = plsc.VectorSubcoreMesh(core_axis_name="c", subcore_axis_name="s")

    @functools.partial(
        pl.kernel,
        mesh=mesh,
        compiler_params=pltpu.CompilerParams(use_tc_tiling_on_sc=False),
        out_type=(
            jax.ShapeDtypeStruct((n_seq, d), jnp.float32),
            jax.ShapeDtypeStruct((n_tgt, d), jnp.float32),
        ),
        scratch_types=[
            pltpu.VMEM((ch,), jnp.int32),
            pltpu.VMEM((ch, d), jnp.float32),
            pltpu.VMEM((tgt_pw,), jnp.int32),
            pltpu.VMEM((tgt_pw, d), jnp.float32),
            pltpu.SemaphoreType.DMA,
        ],
    )
    def gather_k(table_hbm, seq_idx_hbm, item_idx_hbm, out_seq_hbm,
                 out_tgt_hbm, idx_v, rows_v, tidx_v, trows_v, sem):
        wid = lax.axis_index("s") * info.num_cores + lax.axis_index("c")
        tbase = wid * tgt_pw
        pltpu.sync_copy(item_idx_hbm.at[pl.ds(tbase, tgt_pw)], tidx_v)
        pltpu.async_copy(table_hbm.at[tidx_v], trows_v, sem).wait()
        pltpu.sync_copy(trows_v, out_tgt_hbm.at[pl.ds(tbase, tgt_pw)])
        sbase = wid * seq_pw
        for c in range(n_ch):
            off = sbase + c * ch
            pltpu.sync_copy(seq_idx_hbm.at[pl.ds(off, ch)], idx_v)
            pltpu.async_copy(table_hbm.at[idx_v], rows_v, sem).wait()
            pltpu.sync_copy(rows_v, out_seq_hbm.at[pl.ds(off, ch)])

    return gather_k(table, seq_idx_t, item_idx)


def _tc_din(seq3, idx_t, tgt, wq, wk, wp, b1, a1, w2, b2, a2, w3):
    ll, bb, d = seq3.shape
    h1n = wq.shape[1]
    h2n = w2.shape[1]
    blk = 128
    grid = (bb // blk,)

    def body(seq_ref, idx_ref, tgt_ref, wq_ref, wk_ref, wp_ref, b1_ref,
             a1_ref, w2_ref, b2_ref, a2_ref, w3_ref, out_ref):
        seq = seq_ref[...]
        q = tgt_ref[...]
        k2 = seq.reshape(ll * blk, d)
        qb = jnp.concatenate([q] * ll, axis=0)
        qw = q @ wq_ref[...]
        pre1 = (
            k2 @ wk_ref[...]
            + (qb * k2) @ wp_ref[...]
            + jnp.concatenate([qw] * ll, axis=0)
            + b1_ref[...]
        )
        h1 = jnp.where(pre1 > 0, pre1, a1_ref[...] * pre1)
        pre2 = h1 @ w2_ref[...] + b2_ref[...]
        h2 = jnp.where(pre2 > 0, pre2, a2_ref[...] * pre2)
        s3 = h2.reshape(ll, blk, h2n)
        scores = jnp.sum(s3 * w3_ref[...][None], axis=-1)
        mask = idx_ref[...] != 0
        scores = jnp.where(mask, scores, jnp.float32(-1e9))
        m = jnp.max(scores, axis=0, keepdims=True)
        e = jnp.exp(scores - m)
        attn = e / jnp.sum(e, axis=0, keepdims=True)
        out_ref[...] = jnp.sum(attn[:, :, None] * seq, axis=0)

    full = lambda shape: pl.BlockSpec(shape, lambda i: tuple(0 for _ in shape))
    return pl.pallas_call(
        body,
        grid=grid,
        in_specs=[
            pl.BlockSpec((ll, blk, d), lambda i: (0, i, 0)),
            pl.BlockSpec((ll, blk), lambda i: (0, i)),
            pl.BlockSpec((blk, d), lambda i: (i, 0)),
            full(wq.shape), full(wk.shape), full(wp.shape),
            full(b1.shape), full(a1.shape), full(w2.shape),
            full(b2.shape), full(a2.shape), full(w3.shape),
        ],
        out_specs=pl.BlockSpec((blk, d), lambda i: (i, 0)),
        out_shape=jax.ShapeDtypeStruct((bb, d), jnp.float32),
    )(seq3, idx_t, tgt, wq, wk, wp, b1, a1, w2, b2, a2, w3)


def kernel(dense_inputs, sparse_inputs, seq_inputs, item_inputs, table,
           W1, b1, a1, W2, b2, a2, W3, b3):
    b, l, _ = seq_inputs.shape
    d = table.shape[1]
    idx_t = seq_inputs[:, :, 0].astype(jnp.int32).T
    item_idx = item_inputs[:, 0].astype(jnp.int32)

    v = table.shape[0]
    table_rm = _sc_transpose(table.T).reshape(v, d)
    seq_rows, tgt_rows = _sc_gather(table_rm, idx_t.reshape(l * b), item_idx)
    seq3 = seq_rows.reshape(l, b, d)

    w1q, w1k, w1d, w1p = W1[:d], W1[d:2 * d], W1[2 * d:3 * d], W1[3 * d:]
    wq = w1q + w1d
    wk = w1k - w1d
    user_info = _tc_din(
        seq3, idx_t, tgt_rows,
        wq, wk, w1p,
        b1.reshape(1, -1), a1.reshape(1, -1),
        W2, b2.reshape(1, -1), a2.reshape(1, -1),
        W3.reshape(1, -1),
    )
    return user_info


# speedup vs baseline: 1.4670x; 1.4670x over previous
import functools

import jax
import jax.numpy as jnp
from jax import lax
from jax.experimental import pallas as pl
from jax.experimental.pallas import tpu as pltpu
from jax.experimental.pallas import tpu_sc as plsc


def _sc_transpose(table_t):
    d, v = table_t.shape
    lanes_per_stripe = 512
    n_stripes = v // lanes_per_stripe
    tail = v - n_stripes * lanes_per_stripe
    info = plsc.get_sparse_core_info()
    nw = info.num_cores * info.num_subcores
    main_steps = n_stripes // nw
    rem_stripes = n_stripes - main_steps * nw
    elems = lanes_per_stripe * d
    n_chunks = lanes_per_stripe // 16

    mesh = plsc.VectorSubcoreMesh(core_axis_name="c", subcore_axis_name="s")

    @functools.partial(
        pl.kernel,
        mesh=mesh,
        compiler_params=pltpu.CompilerParams(use_tc_tiling_on_sc=True,
                                             needs_layout_passes=False),
        out_type=jax.ShapeDtypeStruct((v * d,), jnp.float32),
        scratch_types=[
            pltpu.VMEM((d, lanes_per_stripe), jnp.float32),
            pltpu.VMEM((d, lanes_per_stripe), jnp.float32),
            pltpu.VMEM((elems,), jnp.float32),
            pltpu.VMEM((elems,), jnp.float32),
            pltpu.VMEM((d, tail), jnp.float32) if tail else None,
            pltpu.SemaphoreType.DMA,
            pltpu.SemaphoreType.DMA,
            pltpu.SemaphoreType.DMA,
            pltpu.SemaphoreType.DMA,
        ],
    )
    def transpose_k(tt_hbm, out_hbm, colbuf0, colbuf1, rowbuf0, rowbuf1,
                    tailbuf, sem_in0, sem_in1, sem_out0, sem_out1):
        colbuf = (colbuf0, colbuf1)
        rowbuf = (rowbuf0, rowbuf1)
        sem_in = (sem_in0, sem_in1)
        sem_out = (sem_out0, sem_out1)
        wid = lax.axis_index("s") * info.num_cores + lax.axis_index("c")
        iota32 = jnp.arange(16, dtype=jnp.int32) * d

        def stripe_of(t):
            return wid + t * nw

        def issue_in(t, b):
            s = stripe_of(t)
            return pltpu.make_async_copy(
                tt_hbm.at[:, pl.ds(s * lanes_per_stripe, lanes_per_stripe)],
                colbuf[b], sem_in[b])

        def issue_out(t, b):
            s = stripe_of(t)
            return pltpu.make_async_copy(
                rowbuf[b], out_hbm.at[pl.ds(s * elems, elems)],
                sem_out[b])

        def compute(cb, rb):
            def d_body(di, _):
                for k in range(n_chunks):
                    vals = cb[di, pl.ds(k * 16, 16)]
                    plsc.store_scatter(rb, [iota32 + (k * 16 * d + di)], vals)
                return 0
            lax.fori_loop(0, d, d_body, 0)

        issue_in(0, 0).start()

        def step(t, b):
            pltpu.make_async_copy(
                tt_hbm.at[:, pl.ds(0, lanes_per_stripe)],
                colbuf[b], sem_in[b]).wait()

            @pl.when(t + 1 < main_steps)
            def _():
                issue_in(t + 1, 1 - b).start()

            @pl.when(t >= 2)
            def _():
                pltpu.make_async_copy(
                    rowbuf[b], out_hbm.at[pl.ds(0, elems)],
                    sem_out[b]).wait()

            compute(colbuf[b], rowbuf[b])
            issue_out(t, b).start()

        def pair_body(u, _):
            step(2 * u, 0)
            step(2 * u + 1, 1)
            return 0

        lax.fori_loop(0, main_steps // 2, pair_body, 0)
        if main_steps % 2:
            step(main_steps - 1, 0)

        for b in range(2):
            if main_steps > 1 - b:
                pltpu.make_async_copy(
                    rowbuf[b], out_hbm.at[pl.ds(0, elems)],
                    sem_out[b]).wait()

        for r in range(rem_stripes):
            @pl.when(wid == r)
            def _():
                s = main_steps * nw + r
                pltpu.sync_copy(
                    tt_hbm.at[:, pl.ds(s * lanes_per_stripe,
                                       lanes_per_stripe)], colbuf[0])
                compute(colbuf[0], rowbuf[0])
                pltpu.sync_copy(rowbuf[0],
                                out_hbm.at[pl.ds(s * elems, elems)])

        if tail:
            @pl.when(wid == rem_stripes)
            def _():
                base = n_stripes * lanes_per_stripe
                pltpu.sync_copy(tt_hbm.at[:, pl.ds(base, tail)], tailbuf)

                def td_body(di, _):
                    for k in range(tail // 16):
                        vals = tailbuf[di, pl.ds(k * 16, 16)]
                        plsc.store_scatter(
                            rowbuf[0], [iota32 + (k * 16 * d + di)], vals)
                    return 0

                lax.fori_loop(0, d, td_body, 0)
                pltpu.sync_copy(rowbuf[0].at[pl.ds(0, tail * d)],
                                out_hbm.at[pl.ds(base * d, tail * d)])

    return transpose_k(table_t)


def _sc_gather(table, seq_idx_t, item_idx):
    info = plsc.get_sparse_core_info()
    nw = info.num_cores * info.num_subcores
    n_seq = seq_idx_t.shape[0]
    n_tgt = item_idx.shape[0]
    d = table.shape[1]
    seq_pw = n_seq // nw
    tgt_pw = n_tgt // nw
    ch = 800
    n_ch = seq_pw // ch

    mesh = plsc.VectorSubcoreMesh(core_axis_name="c", subcore_axis_name="s")

    @functools.partial(
        pl.kernel,
        mesh=mesh,
        compiler_params=pltpu.CompilerParams(use_tc_tiling_on_sc=False),
        out_type=(
            jax.ShapeDtypeStruct((n_seq, d), jnp.float32),
            jax.ShapeDtypeStruct((n_tgt, d), jnp.float32),
        ),
        scratch_types=[
            pltpu.VMEM((ch,), jnp.int32),
            pltpu.VMEM((ch, d), jnp.float32),
            pltpu.VMEM((tgt_pw,), jnp.int32),
            pltpu.VMEM((tgt_pw, d), jnp.float32),
            pltpu.SemaphoreType.DMA,
        ],
    )
    def gather_k(table_hbm, seq_idx_hbm, item_idx_hbm, out_seq_hbm,
                 out_tgt_hbm, idx_v, rows_v, tidx_v, trows_v, sem):
        wid = lax.axis_index("s") * info.num_cores + lax.axis_index("c")
        tbase = wid * tgt_pw
        pltpu.sync_copy(item_idx_hbm.at[pl.ds(tbase, tgt_pw)], tidx_v)
        pltpu.async_copy(table_hbm.at[tidx_v], trows_v, sem).wait()
        pltpu.sync_copy(trows_v, out_tgt_hbm.at[pl.ds(tbase, tgt_pw)])
        sbase = wid * seq_pw
        for c in range(n_ch):
            off = sbase + c * ch
            pltpu.sync_copy(seq_idx_hbm.at[pl.ds(off, ch)], idx_v)
            pltpu.async_copy(table_hbm.at[idx_v], rows_v, sem).wait()
            pltpu.sync_copy(rows_v, out_seq_hbm.at[pl.ds(off, ch)])

    return gather_k(table, seq_idx_t, item_idx)


def _tc_din(seq3, idx_t, tgt, wq, wk, wp, b1, a1, w2, b2, a2, w3):
    ll, bb, d = seq3.shape
    h1n = wq.shape[1]
    h2n = w2.shape[1]
    blk = 128
    grid = (bb // blk,)

    def body(seq_ref, idx_ref, tgt_ref, wq_ref, wk_ref, wp_ref, b1_ref,
             a1_ref, w2_ref, b2_ref, a2_ref, w3_ref, out_ref):
        seq = seq_ref[...]
        q = tgt_ref[...]
        k2 = seq.reshape(ll * blk, d)
        qb = jnp.concatenate([q] * ll, axis=0)
        qw = q @ wq_ref[...]
        pre1 = (
            k2 @ wk_ref[...]
            + (qb * k2) @ wp_ref[...]
            + jnp.concatenate([qw] * ll, axis=0)
            + b1_ref[...]
        )
        h1 = jnp.where(pre1 > 0, pre1, a1_ref[...] * pre1)
        pre2 = h1 @ w2_ref[...] + b2_ref[...]
        h2 = jnp.where(pre2 > 0, pre2, a2_ref[...] * pre2)
        s3 = h2.reshape(ll, blk, h2n)
        scores = jnp.sum(s3 * w3_ref[...][None], axis=-1)
        mask = idx_ref[...] != 0
        scores = jnp.where(mask, scores, jnp.float32(-1e9))
        m = jnp.max(scores, axis=0, keepdims=True)
        e = jnp.exp(scores - m)
        attn = e / jnp.sum(e, axis=0, keepdims=True)
        out_ref[...] = jnp.sum(attn[:, :, None] * seq, axis=0)

    full = lambda shape: pl.BlockSpec(shape, lambda i: tuple(0 for _ in shape))
    return pl.pallas_call(
        body,
        grid=grid,
        in_specs=[
            pl.BlockSpec((ll, blk, d), lambda i: (0, i, 0)),
            pl.BlockSpec((ll, blk), lambda i: (0, i)),
            pl.BlockSpec((blk, d), lambda i: (i, 0)),
            full(wq.shape), full(wk.shape), full(wp.shape),
            full(b1.shape), full(a1.shape), full(w2.shape),
            full(b2.shape), full(a2.shape), full(w3.shape),
        ],
        out_specs=pl.BlockSpec((blk, d), lambda i: (i, 0)),
        out_shape=jax.ShapeDtypeStruct((bb, d), jnp.float32),
    )(seq3, idx_t, tgt, wq, wk, wp, b1, a1, w2, b2, a2, w3)


def kernel(dense_inputs, sparse_inputs, seq_inputs, item_inputs, table,
           W1, b1, a1, W2, b2, a2, W3, b3):
    b, l, _ = seq_inputs.shape
    d = table.shape[1]
    idx_t = seq_inputs[:, :, 0].astype(jnp.int32).T
    item_idx = item_inputs[:, 0].astype(jnp.int32)

    v = table.shape[0]
    table_rm = _sc_transpose(table.T).reshape(v, d)
    seq_rows, tgt_rows = _sc_gather(table_rm, idx_t.reshape(l * b), item_idx)
    seq3 = seq_rows.reshape(l, b, d)

    w1q, w1k, w1d, w1p = W1[:d], W1[d:2 * d], W1[2 * d:3 * d], W1[3 * d:]
    wq = w1q + w1d
    wk = w1k - w1d
    user_info = _tc_din(
        seq3, idx_t, tgt_rows,
        wq, wk, w1p,
        b1.reshape(1, -1), a1.reshape(1, -1),
        W2, b2.reshape(1, -1), a2.reshape(1, -1),
        W3.reshape(1, -1),
    )
    return user_info


# speedup vs baseline: 1.7610x; 1.2004x over previous
import functools

import jax
import jax.numpy as jnp
from jax import lax
from jax.experimental import pallas as pl
from jax.experimental.pallas import tpu as pltpu
from jax.experimental.pallas import tpu_sc as plsc


def _sc_transpose(table_t):
    d, v = table_t.shape
    pitch = d + 1
    lanes = 512
    n_stripes = v // lanes
    tail = v - n_stripes * lanes
    info = plsc.get_sparse_core_info()
    nw = info.num_cores * info.num_subcores
    main_steps = n_stripes // nw
    rem_stripes = n_stripes - main_steps * nw
    elems = lanes * d
    n_chunks = lanes // 16

    mesh = plsc.VectorSubcoreMesh(core_axis_name="c", subcore_axis_name="s")

    @functools.partial(
        pl.kernel,
        mesh=mesh,
        compiler_params=pltpu.CompilerParams(use_tc_tiling_on_sc=True,
                                             needs_layout_passes=False),
        out_type=jax.ShapeDtypeStruct((v * d,), jnp.float32),
        scratch_types=[
            pltpu.VMEM((d, lanes), jnp.float32),
            pltpu.VMEM((d, lanes), jnp.float32),
            pltpu.VMEM((lanes * pitch,), jnp.float32),
            pltpu.VMEM((elems,), jnp.float32),
            pltpu.VMEM((elems,), jnp.float32),
            pltpu.VMEM((d, tail), jnp.float32) if tail else None,
            pltpu.SemaphoreType.DMA,
            pltpu.SemaphoreType.DMA,
            pltpu.SemaphoreType.DMA,
            pltpu.SemaphoreType.DMA,
        ],
    )
    def transpose_k(tt_hbm, out_hbm, colbuf0, colbuf1, stage, rowbuf0,
                    rowbuf1, tailbuf, sem_in0, sem_in1, sem_out0, sem_out1):
        colbuf = (colbuf0, colbuf1)
        rowbuf = (rowbuf0, rowbuf1)
        sem_in = (sem_in0, sem_in1)
        sem_out = (sem_out0, sem_out1)
        wid = lax.axis_index("s") * info.num_cores + lax.axis_index("c")
        iota = jnp.arange(16, dtype=jnp.int32)
        iotap = iota * pitch

        def issue_in(t, b):
            s = wid + t * nw
            return pltpu.make_async_copy(
                tt_hbm.at[:, pl.ds(s * lanes, lanes)], colbuf[b], sem_in[b])

        def issue_out(t, b):
            s = wid + t * nw
            return pltpu.make_async_copy(
                rowbuf[b], out_hbm.at[pl.ds(s * elems, elems)], sem_out[b])

        def transpose_block(cb, rb, n_rows):
            def d_body(di, _):
                for k in range(n_rows // 16):
                    vals = cb[di, pl.ds(k * 16, 16)]
                    plsc.store_scatter(
                        stage, [iotap + (k * 16 * pitch + di)], vals)
                return 0
            lax.fori_loop(0, d, d_body, 0)

            def r_body(i, _):
                for h in range(2):
                    for half in range(d // 16):
                        vals = plsc.load_gather(
                            stage, [iota + ((i * 2 + h) * pitch + half * 16)])
                        rb[pl.ds((i * 2 + h) * d + half * 16, 16)] = vals
                return 0
            lax.fori_loop(0, n_rows // 2, r_body, 0)

        issue_in(0, 0).start()

        def step(t, b):
            pltpu.make_async_copy(
                tt_hbm.at[:, pl.ds(0, lanes)], colbuf[b], sem_in[b]).wait()

            @pl.when(t + 1 < main_steps)
            def _():
                issue_in(t + 1, 1 - b).start()

            @pl.when(t >= 2)
            def _():
                pltpu.make_async_copy(
                    rowbuf[b], out_hbm.at[pl.ds(0, elems)], sem_out[b]).wait()

            transpose_block(colbuf[b], rowbuf[b], lanes)
            issue_out(t, b).start()

        def pair_body(u, _):
            step(2 * u, 0)
            step(2 * u + 1, 1)
            return 0

        lax.fori_loop(0, main_steps // 2, pair_body, 0)
        if main_steps % 2:
            step(main_steps - 1, 0)

        for b in range(2):
            if main_steps > 1 - b:
                pltpu.make_async_copy(
                    rowbuf[b], out_hbm.at[pl.ds(0, elems)], sem_out[b]).wait()

        for r in range(rem_stripes):
            @pl.when(wid == r)
            def _():
                s = main_steps * nw + r
                pltpu.sync_copy(tt_hbm.at[:, pl.ds(s * lanes, lanes)],
                                colbuf[0])
                transpose_block(colbuf[0], rowbuf[0], lanes)
                pltpu.sync_copy(rowbuf[0],
                                out_hbm.at[pl.ds(s * elems, elems)])

        if tail:
            @pl.when(wid == rem_stripes)
            def _():
                base = n_stripes * lanes
                pltpu.sync_copy(tt_hbm.at[:, pl.ds(base, tail)], tailbuf)
                transpose_block(tailbuf, rowbuf[0], tail)
                pltpu.sync_copy(rowbuf[0].at[pl.ds(0, tail * d)],
                                out_hbm.at[pl.ds(base * d, tail * d)])

    return transpose_k(table_t)


def _sc_gather(table, seq_idx_t, item_idx):
    info = plsc.get_sparse_core_info()
    nw = info.num_cores * info.num_subcores
    n_seq = seq_idx_t.shape[0]
    n_tgt = item_idx.shape[0]
    d = table.shape[1]
    seq_pw = n_seq // nw
    tgt_pw = n_tgt // nw
    ch = 800
    n_ch = seq_pw // ch

    mesh = plsc.VectorSubcoreMesh(core_axis_name="c", subcore_axis_name="s")

    @functools.partial(
        pl.kernel,
        mesh=mesh,
        compiler_params=pltpu.CompilerParams(use_tc_tiling_on_sc=False),
        out_type=(
            jax.ShapeDtypeStruct((n_seq, d), jnp.float32),
            jax.ShapeDtypeStruct((n_tgt, d), jnp.float32),
        ),
        scratch_types=[
            pltpu.VMEM((ch,), jnp.int32),
            pltpu.VMEM((ch, d), jnp.float32),
            pltpu.VMEM((tgt_pw,), jnp.int32),
            pltpu.VMEM((tgt_pw, d), jnp.float32),
            pltpu.SemaphoreType.DMA,
        ],
    )
    def gather_k(table_hbm, seq_idx_hbm, item_idx_hbm, out_seq_hbm,
                 out_tgt_hbm, idx_v, rows_v, tidx_v, trows_v, sem):
        wid = lax.axis_index("s") * info.num_cores + lax.axis_index("c")
        tbase = wid * tgt_pw
        pltpu.sync_copy(item_idx_hbm.at[pl.ds(tbase, tgt_pw)], tidx_v)
        pltpu.async_copy(table_hbm.at[tidx_v], trows_v, sem).wait()
        pltpu.sync_copy(trows_v, out_tgt_hbm.at[pl.ds(tbase, tgt_pw)])
        sbase = wid * seq_pw
        for c in range(n_ch):
            off = sbase + c * ch
            pltpu.sync_copy(seq_idx_hbm.at[pl.ds(off, ch)], idx_v)
            pltpu.async_copy(table_hbm.at[idx_v], rows_v, sem).wait()
            pltpu.sync_copy(rows_v, out_seq_hbm.at[pl.ds(off, ch)])

    return gather_k(table, seq_idx_t, item_idx)


def _tc_din(seq3, idx_t, tgt, wq, wk, wp, b1, a1, w2, b2, a2, w3):
    ll, bb, d = seq3.shape
    h2n = w2.shape[1]
    blk = 128
    grid = (bb // blk,)

    def body(seq_ref, idx_ref, tgt_ref, wq_ref, wk_ref, wp_ref, b1_ref,
             a1_ref, w2_ref, b2_ref, a2_ref, w3_ref, out_ref):
        seq = seq_ref[...]
        q = tgt_ref[...]
        k2 = seq.reshape(ll * blk, d)
        qb = jnp.concatenate([q] * ll, axis=0)
        qw = q @ wq_ref[...]
        pre1 = (
            k2 @ wk_ref[...]
            + (qb * k2) @ wp_ref[...]
            + jnp.concatenate([qw] * ll, axis=0)
            + b1_ref[...]
        )
        h1 = jnp.where(pre1 > 0, pre1, a1_ref[...] * pre1)
        pre2 = h1 @ w2_ref[...] + b2_ref[...]
        h2 = jnp.where(pre2 > 0, pre2, a2_ref[...] * pre2)
        s3 = h2.reshape(ll, blk, h2n)
        scores = jnp.sum(s3 * w3_ref[...][None], axis=-1)
        mask = idx_ref[...] != 0
        scores = jnp.where(mask, scores, jnp.float32(-1e9))
        m = jnp.max(scores, axis=0, keepdims=True)
        e = jnp.exp(scores - m)
        attn = e / jnp.sum(e, axis=0, keepdims=True)
        out_ref[...] = jnp.sum(attn[:, :, None] * seq, axis=0)

    full = lambda shape: pl.BlockSpec(shape, lambda i: tuple(0 for _ in shape))
    return pl.pallas_call(
        body,
        grid=grid,
        in_specs=[
            pl.BlockSpec((ll, blk, d), lambda i: (0, i, 0)),
            pl.BlockSpec((ll, blk), lambda i: (0, i)),
            pl.BlockSpec((blk, d), lambda i: (i, 0)),
            full(wq.shape), full(wk.shape), full(wp.shape),
            full(b1.shape), full(a1.shape), full(w2.shape),
            full(b2.shape), full(a2.shape), full(w3.shape),
        ],
        out_specs=pl.BlockSpec((blk, d), lambda i: (i, 0)),
        out_shape=jax.ShapeDtypeStruct((bb, d), jnp.float32),
    )(seq3, idx_t, tgt, wq, wk, wp, b1, a1, w2, b2, a2, w3)


def kernel(dense_inputs, sparse_inputs, seq_inputs, item_inputs, table,
           W1, b1, a1, W2, b2, a2, W3, b3):
    b, l, _ = seq_inputs.shape
    d = table.shape[1]
    idx_t = seq_inputs[:, :, 0].astype(jnp.int32).T
    item_idx = item_inputs[:, 0].astype(jnp.int32)

    v = table.shape[0]
    table_rm = _sc_transpose(table.T).reshape(v, d)
    seq_rows, tgt_rows = _sc_gather(table_rm, idx_t.reshape(l * b), item_idx)
    seq3 = seq_rows.reshape(l, b, d)

    w1q, w1k, w1d, w1p = W1[:d], W1[d:2 * d], W1[2 * d:3 * d], W1[3 * d:]
    wq = w1q + w1d
    wk = w1k - w1d
    user_info = _tc_din(
        seq3, idx_t, tgt_rows,
        wq, wk, w1p,
        b1.reshape(1, -1), a1.reshape(1, -1),
        W2, b2.reshape(1, -1), a2.reshape(1, -1),
        W3.reshape(1, -1),
    )
    return user_info


# speedup vs baseline: 2.9721x; 1.6878x over previous
import functools

import jax
import jax.numpy as jnp
from jax import lax
from jax.experimental import pallas as pl
from jax.experimental.pallas import tpu as pltpu
from jax.experimental.pallas import tpu_sc as plsc


def _sc_transpose(table_t):
    d, v = table_t.shape
    pitch = d + 1
    lanes = 512
    n_stripes = v // lanes
    tail = v - n_stripes * lanes
    info = plsc.get_sparse_core_info()
    nw = info.num_cores * info.num_subcores
    main_steps = n_stripes // nw
    rem_stripes = n_stripes - main_steps * nw
    elems = lanes * d
    n_chunks = lanes // 16

    mesh = plsc.VectorSubcoreMesh(core_axis_name="c", subcore_axis_name="s")

    @functools.partial(
        pl.kernel,
        mesh=mesh,
        compiler_params=pltpu.CompilerParams(use_tc_tiling_on_sc=True,
                                             needs_layout_passes=False),
        out_type=jax.ShapeDtypeStruct((v * d,), jnp.float32),
        scratch_types=[
            pltpu.VMEM((d, lanes), jnp.float32),
            pltpu.VMEM((d, lanes), jnp.float32),
            pltpu.VMEM((lanes * pitch,), jnp.float32),
            pltpu.VMEM((elems,), jnp.float32),
            pltpu.VMEM((elems,), jnp.float32),
            pltpu.VMEM((d, tail), jnp.float32) if tail else None,
            pltpu.SemaphoreType.DMA,
            pltpu.SemaphoreType.DMA,
            pltpu.SemaphoreType.DMA,
            pltpu.SemaphoreType.DMA,
        ],
    )
    def transpose_k(tt_hbm, out_hbm, colbuf0, colbuf1, stage, rowbuf0,
                    rowbuf1, tailbuf, sem_in0, sem_in1, sem_out0, sem_out1):
        colbuf = (colbuf0, colbuf1)
        rowbuf = (rowbuf0, rowbuf1)
        sem_in = (sem_in0, sem_in1)
        sem_out = (sem_out0, sem_out1)
        wid = lax.axis_index("s") * info.num_cores + lax.axis_index("c")
        iota = jnp.arange(16, dtype=jnp.int32)
        iotap = iota * pitch

        def issue_in(t, b):
            s = wid + t * nw
            return pltpu.make_async_copy(
                tt_hbm.at[:, pl.ds(s * lanes, lanes)], colbuf[b], sem_in[b])

        def issue_out(t, b):
            s = wid + t * nw
            return pltpu.make_async_copy(
                rowbuf[b], out_hbm.at[pl.ds(s * elems, elems)], sem_out[b])

        def transpose_block(cb, rb, n_rows):
            def d_body(di, _):
                for k0 in range(0, n_rows // 16, 4):
                    vs = [cb[di, pl.ds((k0 + j) * 16, 16)] for j in range(4)]
                    for j in range(4):
                        plsc.store_scatter(
                            stage,
                            [iotap + ((k0 + j) * 16 * pitch + di)], vs[j])
                return 0
            lax.fori_loop(0, d, d_body, 0)

            def r_body(i, _):
                base = i * 2
                srcs = [(h, half) for h in range(2) for half in range(d // 16)]
                vs = [plsc.load_gather(
                    stage, [iota + ((base + h) * pitch + half * 16)])
                    for (h, half) in srcs]
                for (h, half), vv in zip(srcs, vs):
                    rb[pl.ds((base + h) * d + half * 16, 16)] = vv
                return 0
            lax.fori_loop(0, n_rows // 2, r_body, 0)

        issue_in(0, 0).start()

        def step(t, b):
            pltpu.make_async_copy(
                tt_hbm.at[:, pl.ds(0, lanes)], colbuf[b], sem_in[b]).wait()

            @pl.when(t + 1 < main_steps)
            def _():
                issue_in(t + 1, 1 - b).start()

            @pl.when(t >= 2)
            def _():
                pltpu.make_async_copy(
                    rowbuf[b], out_hbm.at[pl.ds(0, elems)], sem_out[b]).wait()

            transpose_block(colbuf[b], rowbuf[b], lanes)
            issue_out(t, b).start()

        def pair_body(u, _):
            step(2 * u, 0)
            step(2 * u + 1, 1)
            return 0

        lax.fori_loop(0, main_steps // 2, pair_body, 0)
        if main_steps % 2:
            step(main_steps - 1, 0)

        for b in range(2):
            if main_steps > 1 - b:
                pltpu.make_async_copy(
                    rowbuf[b], out_hbm.at[pl.ds(0, elems)], sem_out[b]).wait()

        for r in range(rem_stripes):
            @pl.when(wid == r)
            def _():
                s = main_steps * nw + r
                pltpu.sync_copy(tt_hbm.at[:, pl.ds(s * lanes, lanes)],
                                colbuf[0])
                transpose_block(colbuf[0], rowbuf[0], lanes)
                pltpu.sync_copy(rowbuf[0],
                                out_hbm.at[pl.ds(s * elems, elems)])

        if tail:
            @pl.when(wid == rem_stripes)
            def _():
                base = n_stripes * lanes
                pltpu.sync_copy(tt_hbm.at[:, pl.ds(base, tail)], tailbuf)
                transpose_block(tailbuf, rowbuf[0], tail)
                pltpu.sync_copy(rowbuf[0].at[pl.ds(0, tail * d)],
                                out_hbm.at[pl.ds(base * d, tail * d)])

    return transpose_k(table_t)


def _sc_gather(table, seq_idx_t, item_idx):
    info = plsc.get_sparse_core_info()
    nw = info.num_cores * info.num_subcores
    n_seq = seq_idx_t.shape[0]
    n_tgt = item_idx.shape[0]
    d = table.shape[1]
    seq_pw = n_seq // nw
    tgt_pw = n_tgt // nw
    ch = 800
    n_ch = seq_pw // ch

    mesh = plsc.VectorSubcoreMesh(core_axis_name="c", subcore_axis_name="s")

    @functools.partial(
        pl.kernel,
        mesh=mesh,
        compiler_params=pltpu.CompilerParams(use_tc_tiling_on_sc=False),
        out_type=(
            jax.ShapeDtypeStruct((n_seq, d), jnp.float32),
            jax.ShapeDtypeStruct((n_tgt, d), jnp.float32),
        ),
        scratch_types=[
            pltpu.VMEM((ch,), jnp.int32),
            pltpu.VMEM((ch, d), jnp.float32),
            pltpu.VMEM((tgt_pw,), jnp.int32),
            pltpu.VMEM((tgt_pw, d), jnp.float32),
            pltpu.SemaphoreType.DMA,
        ],
    )
    def gather_k(table_hbm, seq_idx_hbm, item_idx_hbm, out_seq_hbm,
                 out_tgt_hbm, idx_v, rows_v, tidx_v, trows_v, sem):
        wid = lax.axis_index("s") * info.num_cores + lax.axis_index("c")
        tbase = wid * tgt_pw
        pltpu.sync_copy(item_idx_hbm.at[pl.ds(tbase, tgt_pw)], tidx_v)
        pltpu.async_copy(table_hbm.at[tidx_v], trows_v, sem).wait()
        pltpu.sync_copy(trows_v, out_tgt_hbm.at[pl.ds(tbase, tgt_pw)])
        sbase = wid * seq_pw
        for c in range(n_ch):
            off = sbase + c * ch
            pltpu.sync_copy(seq_idx_hbm.at[pl.ds(off, ch)], idx_v)
            pltpu.async_copy(table_hbm.at[idx_v], rows_v, sem).wait()
            pltpu.sync_copy(rows_v, out_seq_hbm.at[pl.ds(off, ch)])

    return gather_k(table, seq_idx_t, item_idx)


def _tc_din(seq3, idx_t, tgt, wq, wk, wp, b1, a1, w2, b2, a2, w3):
    ll, bb, d = seq3.shape
    h2n = w2.shape[1]
    blk = 128
    grid = (bb // blk,)

    def body(seq_ref, idx_ref, tgt_ref, wq_ref, wk_ref, wp_ref, b1_ref,
             a1_ref, w2_ref, b2_ref, a2_ref, w3_ref, out_ref):
        seq = seq_ref[...]
        q = tgt_ref[...]
        k2 = seq.reshape(ll * blk, d)
        qb = jnp.concatenate([q] * ll, axis=0)
        qw = q @ wq_ref[...]
        pre1 = (
            k2 @ wk_ref[...]
            + (qb * k2) @ wp_ref[...]
            + jnp.concatenate([qw] * ll, axis=0)
            + b1_ref[...]
        )
        h1 = jnp.where(pre1 > 0, pre1, a1_ref[...] * pre1)
        pre2 = h1 @ w2_ref[...] + b2_ref[...]
        h2 = jnp.where(pre2 > 0, pre2, a2_ref[...] * pre2)
        s3 = h2.reshape(ll, blk, h2n)
        scores = jnp.sum(s3 * w3_ref[...][None], axis=-1)
        mask = idx_ref[...] != 0
        scores = jnp.where(mask, scores, jnp.float32(-1e9))
        m = jnp.max(scores, axis=0, keepdims=True)
        e = jnp.exp(scores - m)
        attn = e / jnp.sum(e, axis=0, keepdims=True)
        out_ref[...] = jnp.sum(attn[:, :, None] * seq, axis=0)

    full = lambda shape: pl.BlockSpec(shape, lambda i: tuple(0 for _ in shape))
    return pl.pallas_call(
        body,
        grid=grid,
        in_specs=[
            pl.BlockSpec((ll, blk, d), lambda i: (0, i, 0)),
            pl.BlockSpec((ll, blk), lambda i: (0, i)),
            pl.BlockSpec((blk, d), lambda i: (i, 0)),
            full(wq.shape), full(wk.shape), full(wp.shape),
            full(b1.shape), full(a1.shape), full(w2.shape),
            full(b2.shape), full(a2.shape), full(w3.shape),
        ],
        out_specs=pl.BlockSpec((blk, d), lambda i: (i, 0)),
        out_shape=jax.ShapeDtypeStruct((bb, d), jnp.float32),
    )(seq3, idx_t, tgt, wq, wk, wp, b1, a1, w2, b2, a2, w3)


def kernel(dense_inputs, sparse_inputs, seq_inputs, item_inputs, table,
           W1, b1, a1, W2, b2, a2, W3, b3):
    b, l, _ = seq_inputs.shape
    d = table.shape[1]
    idx_t = seq_inputs[:, :, 0].astype(jnp.int32).T
    item_idx = item_inputs[:, 0].astype(jnp.int32)

    v = table.shape[0]
    table_rm = _sc_transpose(table.T).reshape(v, d)
    seq_rows, tgt_rows = _sc_gather(table_rm, idx_t.reshape(l * b), item_idx)
    seq3 = seq_rows.reshape(l, b, d)

    w1q, w1k, w1d, w1p = W1[:d], W1[d:2 * d], W1[2 * d:3 * d], W1[3 * d:]
    wq = w1q + w1d
    wk = w1k - w1d
    user_info = _tc_din(
        seq3, idx_t, tgt_rows,
        wq, wk, w1p,
        b1.reshape(1, -1), a1.reshape(1, -1),
        W2, b2.reshape(1, -1), a2.reshape(1, -1),
        W3.reshape(1, -1),
    )
    return user_info


# speedup vs baseline: 3.8562x; 1.2974x over previous
import functools

import jax
import jax.numpy as jnp
from jax import lax
from jax.experimental import pallas as pl
from jax.experimental.pallas import tpu as pltpu
from jax.experimental.pallas import tpu_sc as plsc


def _sc_transpose(table_t):
    d, v = table_t.shape
    pitch = d + 1
    lanes = 512
    n_stripes = v // lanes
    tail = v - n_stripes * lanes
    info = plsc.get_sparse_core_info()
    nw = info.num_cores * info.num_subcores
    main_steps = n_stripes // nw
    rem_stripes = n_stripes - main_steps * nw
    elems = lanes * d
    n_chunks = lanes // 16

    mesh = plsc.VectorSubcoreMesh(core_axis_name="c", subcore_axis_name="s")

    @functools.partial(
        pl.kernel,
        mesh=mesh,
        compiler_params=pltpu.CompilerParams(use_tc_tiling_on_sc=True,
                                             needs_layout_passes=False),
        out_type=jax.ShapeDtypeStruct((v * d,), jnp.float32),
        scratch_types=[
            pltpu.VMEM((d, lanes), jnp.float32),
            pltpu.VMEM((d, lanes), jnp.float32),
            pltpu.VMEM((lanes * pitch,), jnp.float32),
            pltpu.VMEM((elems,), jnp.float32),
            pltpu.VMEM((elems,), jnp.float32),
            pltpu.VMEM((d, tail), jnp.float32) if tail else None,
            pltpu.SemaphoreType.DMA,
            pltpu.SemaphoreType.DMA,
            pltpu.SemaphoreType.DMA,
            pltpu.SemaphoreType.DMA,
        ],
    )
    def transpose_k(tt_hbm, out_hbm, colbuf0, colbuf1, stage, rowbuf0,
                    rowbuf1, tailbuf, sem_in0, sem_in1, sem_out0, sem_out1):
        colbuf = (colbuf0, colbuf1)
        rowbuf = (rowbuf0, rowbuf1)
        sem_in = (sem_in0, sem_in1)
        sem_out = (sem_out0, sem_out1)
        wid = lax.axis_index("s") * info.num_cores + lax.axis_index("c")
        iota = jnp.arange(16, dtype=jnp.int32)
        iotap = iota * pitch

        def issue_in(t, b):
            s = wid + t * nw
            return pltpu.make_async_copy(
                tt_hbm.at[:, pl.ds(s * lanes, lanes)], colbuf[b], sem_in[b])

        def issue_out(t, b):
            s = wid + t * nw
            return pltpu.make_async_copy(
                rowbuf[b], out_hbm.at[pl.ds(s * elems, elems)], sem_out[b])

        def transpose_block(cb, rb, n_rows):
            def d_body(di, _):
                for k0 in range(0, n_rows // 16, 4):
                    vs = [cb[di, pl.ds((k0 + j) * 16, 16)] for j in range(4)]
                    for j in range(4):
                        plsc.store_scatter(
                            stage,
                            [iotap + ((k0 + j) * 16 * pitch + di)], vs[j])
                return 0
            lax.fori_loop(0, d, d_body, 0)

            def r_body(i, _):
                base = i * 2
                srcs = [(h, half) for h in range(2) for half in range(d // 16)]
                vs = [plsc.load_gather(
                    stage, [iota + ((base + h) * pitch + half * 16)])
                    for (h, half) in srcs]
                for (h, half), vv in zip(srcs, vs):
                    rb[pl.ds((base + h) * d + half * 16, 16)] = vv
                return 0
            lax.fori_loop(0, n_rows // 2, r_body, 0)

        issue_in(0, 0).start()

        def step(t, b):
            pltpu.make_async_copy(
                tt_hbm.at[:, pl.ds(0, lanes)], colbuf[b], sem_in[b]).wait()

            @pl.when(t + 1 < main_steps)
            def _():
                issue_in(t + 1, 1 - b).start()

            @pl.when(t >= 2)
            def _():
                pltpu.make_async_copy(
                    rowbuf[b], out_hbm.at[pl.ds(0, elems)], sem_out[b]).wait()

            transpose_block(colbuf[b], rowbuf[b], lanes)
            issue_out(t, b).start()

        def pair_body(u, _):
            step(2 * u, 0)
            step(2 * u + 1, 1)
            return 0

        lax.fori_loop(0, main_steps // 2, pair_body, 0)
        if main_steps % 2:
            step(main_steps - 1, 0)

        for b in range(2):
            if main_steps > 1 - b:
                pltpu.make_async_copy(
                    rowbuf[b], out_hbm.at[pl.ds(0, elems)], sem_out[b]).wait()

        for r in range(rem_stripes):
            @pl.when(wid == r)
            def _():
                s = main_steps * nw + r
                pltpu.sync_copy(tt_hbm.at[:, pl.ds(s * lanes, lanes)],
                                colbuf[0])
                transpose_block(colbuf[0], rowbuf[0], lanes)
                pltpu.sync_copy(rowbuf[0],
                                out_hbm.at[pl.ds(s * elems, elems)])

        if tail:
            @pl.when(wid == rem_stripes)
            def _():
                base = n_stripes * lanes
                pltpu.sync_copy(tt_hbm.at[:, pl.ds(base, tail)], tailbuf)
                transpose_block(tailbuf, rowbuf[0], tail)
                pltpu.sync_copy(rowbuf[0].at[pl.ds(0, tail * d)],
                                out_hbm.at[pl.ds(base * d, tail * d)])

    return transpose_k(table_t)


def _sc_gather(table, seq_idx_t, item_idx):
    info = plsc.get_sparse_core_info()
    nw = info.num_cores * info.num_subcores
    n_seq = seq_idx_t.shape[0]
    n_tgt = item_idx.shape[0]
    d = table.shape[1]
    seq_pw = n_seq // nw
    tgt_pw = n_tgt // nw
    ch = 800
    n_ch = seq_pw // ch

    mesh = plsc.VectorSubcoreMesh(core_axis_name="c", subcore_axis_name="s")

    @functools.partial(
        pl.kernel,
        mesh=mesh,
        compiler_params=pltpu.CompilerParams(use_tc_tiling_on_sc=False),
        out_type=(
            jax.ShapeDtypeStruct((n_seq, d), jnp.float32),
            jax.ShapeDtypeStruct((n_tgt, d), jnp.float32),
        ),
        scratch_types=[
            pltpu.VMEM((ch,), jnp.int32),
            pltpu.VMEM((ch, d), jnp.float32),
            pltpu.VMEM((tgt_pw,), jnp.int32),
            pltpu.VMEM((tgt_pw, d), jnp.float32),
            pltpu.SemaphoreType.DMA,
        ],
    )
    def gather_k(table_hbm, seq_idx_hbm, item_idx_hbm, out_seq_hbm,
                 out_tgt_hbm, idx_v, rows_v, tidx_v, trows_v, sem):
        wid = lax.axis_index("s") * info.num_cores + lax.axis_index("c")
        tbase = wid * tgt_pw
        pltpu.sync_copy(item_idx_hbm.at[pl.ds(tbase, tgt_pw)], tidx_v)
        pltpu.async_copy(table_hbm.at[tidx_v], trows_v, sem).wait()
        pltpu.sync_copy(trows_v, out_tgt_hbm.at[pl.ds(tbase, tgt_pw)])
        sbase = wid * seq_pw
        for c in range(n_ch):
            off = sbase + c * ch
            pltpu.sync_copy(seq_idx_hbm.at[pl.ds(off, ch)], idx_v)
            pltpu.async_copy(table_hbm.at[idx_v], rows_v, sem).wait()
            pltpu.sync_copy(rows_v, out_seq_hbm.at[pl.ds(off, ch)])

    return gather_k(table, seq_idx_t, item_idx)


def _tc_din(seqp, idxp, tgtp, wqbd, wkbd, wpbd, b1t, a1t, w2bd, b2t, a2t,
            w3bd, e4, pack):
    ll, gb, dp = seqp.shape
    gblk = 32
    grid = (gb // gblk,)

    def body(seq_ref, idx_ref, tgt_ref, wq_ref, wk_ref, wp_ref, b1_ref,
             a1_ref, w2_ref, b2_ref, a2_ref, w3_ref, e4_ref, out_ref):
        seq = seq_ref[...]
        k2 = seq.reshape(ll * gblk, dp)
        qp = tgt_ref[...]
        qb = jnp.concatenate([qp] * ll, axis=0)
        qw = qp @ wq_ref[...] + b1_ref[...]
        pre1 = (
            k2 @ wk_ref[...]
            + (qb * k2) @ wp_ref[...]
            + jnp.concatenate([qw] * ll, axis=0)
        )
        h1 = jnp.where(pre1 > 0, pre1, a1_ref[...] * pre1)
        pre2 = h1 @ w2_ref[...] + b2_ref[...]
        h2 = jnp.where(pre2 > 0, pre2, a2_ref[...] * pre2)
        sc2 = h2 @ w3_ref[...]
        sc3 = sc2.reshape(ll, gblk, pack)
        mask = idx_ref[...] != 0
        scores = jnp.where(mask, sc3, jnp.float32(-1e9))
        m = jnp.max(scores, axis=0, keepdims=True)
        e = jnp.exp(scores - m)
        attn = e / jnp.sum(e, axis=0, keepdims=True)
        attnp = (attn.reshape(ll * gblk, pack) @ e4_ref[...])
        out_ref[...] = jnp.sum(attnp.reshape(ll, gblk, dp) * seq, axis=0)

    full = lambda shape: pl.BlockSpec(shape, lambda i: tuple(0 for _ in shape))
    return pl.pallas_call(
        body,
        grid=grid,
        in_specs=[
            pl.BlockSpec((ll, gblk, dp), lambda i: (0, i, 0)),
            pl.BlockSpec((ll, gblk, pack), lambda i: (0, i, 0)),
            pl.BlockSpec((gblk, dp), lambda i: (i, 0)),
            full(wqbd.shape), full(wkbd.shape), full(wpbd.shape),
            full(b1t.shape), full(a1t.shape), full(w2bd.shape),
            full(b2t.shape), full(a2t.shape), full(w3bd.shape),
            full(e4.shape),
        ],
        out_specs=pl.BlockSpec((gblk, dp), lambda i: (i, 0)),
        out_shape=jax.ShapeDtypeStruct((gb, dp), jnp.float32),
    )(seqp, idxp, tgtp, wqbd, wkbd, wpbd, b1t, a1t, w2bd, b2t, a2t, w3bd, e4)


def kernel(dense_inputs, sparse_inputs, seq_inputs, item_inputs, table,
           W1, b1, a1, W2, b2, a2, W3, b3):
    b, l, _ = seq_inputs.shape
    d = table.shape[1]
    idx_t = seq_inputs[:, :, 0].astype(jnp.int32).T
    item_idx = item_inputs[:, 0].astype(jnp.int32)

    v = table.shape[0]
    table_rm = _sc_transpose(table.T).reshape(v, d)
    seq_rows, tgt_rows = _sc_gather(table_rm, idx_t.reshape(l * b), item_idx)

    pack = 128 // d
    seqp = seq_rows.reshape(l, b // pack, pack * d)
    idxp = idx_t.reshape(l, b // pack, pack)
    tgtp = tgt_rows.reshape(b // pack, pack * d)

    w1q, w1k, w1d, w1p = W1[:d], W1[d:2 * d], W1[2 * d:3 * d], W1[3 * d:]
    eye = jnp.eye(pack, dtype=jnp.float32)
    bd = lambda w: jnp.kron(eye, w)
    tile = lambda x: jnp.tile(x, pack).reshape(1, -1)
    user_info = _tc_din(
        seqp, idxp, tgtp,
        bd(w1q + w1d), bd(w1k - w1d), bd(w1p),
        tile(b1), tile(a1),
        bd(W2), tile(b2), tile(a2),
        bd(W3),
        jnp.kron(eye, jnp.ones((1, d), jnp.float32)),
        pack,
    )
    return user_info.reshape(b, d)


# speedup vs baseline: 3.9809x; 1.0324x over previous
import functools

import jax
import jax.numpy as jnp
from jax import lax
from jax.experimental import pallas as pl
from jax.experimental.pallas import tpu as pltpu
from jax.experimental.pallas import tpu_sc as plsc


def _sc_transpose(table_t):
    d, v = table_t.shape
    pitch = d + 1
    lanes = 512
    n_stripes = v // lanes
    tail = v - n_stripes * lanes
    info = plsc.get_sparse_core_info()
    nw = info.num_cores * info.num_subcores
    main_steps = n_stripes // nw
    rem_stripes = n_stripes - main_steps * nw
    elems = lanes * d
    n_chunks = lanes // 16

    mesh = plsc.VectorSubcoreMesh(core_axis_name="c", subcore_axis_name="s")

    @functools.partial(
        pl.kernel,
        mesh=mesh,
        compiler_params=pltpu.CompilerParams(use_tc_tiling_on_sc=True,
                                             needs_layout_passes=False),
        out_type=jax.ShapeDtypeStruct((v * d,), jnp.float32),
        scratch_types=[
            pltpu.VMEM((d, lanes), jnp.float32),
            pltpu.VMEM((d, lanes), jnp.float32),
            pltpu.VMEM((lanes * pitch,), jnp.float32),
            pltpu.VMEM((elems,), jnp.float32),
            pltpu.VMEM((elems,), jnp.float32),
            pltpu.VMEM((d, tail), jnp.float32) if tail else None,
            pltpu.SemaphoreType.DMA,
            pltpu.SemaphoreType.DMA,
            pltpu.SemaphoreType.DMA,
            pltpu.SemaphoreType.DMA,
        ],
    )
    def transpose_k(tt_hbm, out_hbm, colbuf0, colbuf1, stage, rowbuf0,
                    rowbuf1, tailbuf, sem_in0, sem_in1, sem_out0, sem_out1):
        colbuf = (colbuf0, colbuf1)
        rowbuf = (rowbuf0, rowbuf1)
        sem_in = (sem_in0, sem_in1)
        sem_out = (sem_out0, sem_out1)
        wid = lax.axis_index("s") * info.num_cores + lax.axis_index("c")
        iota = jnp.arange(16, dtype=jnp.int32)
        iotap = iota * pitch

        def issue_in(t, b):
            s = wid + t * nw
            return pltpu.make_async_copy(
                tt_hbm.at[:, pl.ds(s * lanes, lanes)], colbuf[b], sem_in[b])

        def issue_out(t, b):
            s = wid + t * nw
            return pltpu.make_async_copy(
                rowbuf[b], out_hbm.at[pl.ds(s * elems, elems)], sem_out[b])

        def transpose_block(cb, rb, n_rows):
            def d_body(di, _):
                for k0 in range(0, n_rows // 16, 4):
                    vs = [cb[di, pl.ds((k0 + j) * 16, 16)] for j in range(4)]
                    for j in range(4):
                        plsc.store_scatter(
                            stage,
                            [iotap + ((k0 + j) * 16 * pitch + di)], vs[j])
                return 0
            lax.fori_loop(0, d, d_body, 0)

            def r_body(i, _):
                base = i * 2
                srcs = [(h, half) for h in range(2) for half in range(d // 16)]
                vs = [plsc.load_gather(
                    stage, [iota + ((base + h) * pitch + half * 16)])
                    for (h, half) in srcs]
                for (h, half), vv in zip(srcs, vs):
                    rb[pl.ds((base + h) * d + half * 16, 16)] = vv
                return 0
            lax.fori_loop(0, n_rows // 2, r_body, 0)

        issue_in(0, 0).start()

        def step(t, b):
            pltpu.make_async_copy(
                tt_hbm.at[:, pl.ds(0, lanes)], colbuf[b], sem_in[b]).wait()

            @pl.when(t + 1 < main_steps)
            def _():
                issue_in(t + 1, 1 - b).start()

            @pl.when(t >= 2)
            def _():
                pltpu.make_async_copy(
                    rowbuf[b], out_hbm.at[pl.ds(0, elems)], sem_out[b]).wait()

            transpose_block(colbuf[b], rowbuf[b], lanes)
            issue_out(t, b).start()

        def pair_body(u, _):
            step(2 * u, 0)
            step(2 * u + 1, 1)
            return 0

        lax.fori_loop(0, main_steps // 2, pair_body, 0)
        if main_steps % 2:
            step(main_steps - 1, 0)

        for b in range(2):
            if main_steps > 1 - b:
                pltpu.make_async_copy(
                    rowbuf[b], out_hbm.at[pl.ds(0, elems)], sem_out[b]).wait()

        for r in range(rem_stripes):
            @pl.when(wid == r)
            def _():
                s = main_steps * nw + r
                pltpu.sync_copy(tt_hbm.at[:, pl.ds(s * lanes, lanes)],
                                colbuf[0])
                transpose_block(colbuf[0], rowbuf[0], lanes)
                pltpu.sync_copy(rowbuf[0],
                                out_hbm.at[pl.ds(s * elems, elems)])

        if tail:
            @pl.when(wid == rem_stripes)
            def _():
                base = n_stripes * lanes
                pltpu.sync_copy(tt_hbm.at[:, pl.ds(base, tail)], tailbuf)
                transpose_block(tailbuf, rowbuf[0], tail)
                pltpu.sync_copy(rowbuf[0].at[pl.ds(0, tail * d)],
                                out_hbm.at[pl.ds(base * d, tail * d)])

    return transpose_k(table_t)


def _sc_gather(table, seq_idx_t, item_idx):
    info = plsc.get_sparse_core_info()
    nw = info.num_cores * info.num_subcores
    n_seq = seq_idx_t.shape[0]
    n_tgt = item_idx.shape[0]
    d = table.shape[1]
    seq_pw = n_seq // nw
    tgt_pw = n_tgt // nw
    ch = 1600
    n_ch = seq_pw // ch

    mesh = plsc.VectorSubcoreMesh(core_axis_name="c", subcore_axis_name="s")

    @functools.partial(
        pl.kernel,
        mesh=mesh,
        compiler_params=pltpu.CompilerParams(use_tc_tiling_on_sc=False),
        out_type=(
            jax.ShapeDtypeStruct((n_seq, d), jnp.float32),
            jax.ShapeDtypeStruct((n_tgt, d), jnp.float32),
        ),
        scratch_types=[
            pltpu.VMEM((ch,), jnp.int32),
            pltpu.VMEM((ch, d), jnp.float32),
            pltpu.VMEM((tgt_pw,), jnp.int32),
            pltpu.VMEM((tgt_pw, d), jnp.float32),
            pltpu.SemaphoreType.DMA,
        ],
    )
    def gather_k(table_hbm, seq_idx_hbm, item_idx_hbm, out_seq_hbm,
                 out_tgt_hbm, idx_v, rows_v, tidx_v, trows_v, sem):
        wid = lax.axis_index("s") * info.num_cores + lax.axis_index("c")
        tbase = wid * tgt_pw
        pltpu.sync_copy(item_idx_hbm.at[pl.ds(tbase, tgt_pw)], tidx_v)
        pltpu.async_copy(table_hbm.at[tidx_v], trows_v, sem).wait()
        pltpu.sync_copy(trows_v, out_tgt_hbm.at[pl.ds(tbase, tgt_pw)])
        sbase = wid * seq_pw
        for c in range(n_ch):
            off = sbase + c * ch
            pltpu.sync_copy(seq_idx_hbm.at[pl.ds(off, ch)], idx_v)
            pltpu.async_copy(table_hbm.at[idx_v], rows_v, sem).wait()
            pltpu.sync_copy(rows_v, out_seq_hbm.at[pl.ds(off, ch)])

    return gather_k(table, seq_idx_t, item_idx)


def _tc_din(seqp, idxp, tgtp, wqbd, wkbd, wpbd, b1t, a1t, w2bd, b2t, a2t,
            w3bd, e4, pack):
    ll, gb, dp = seqp.shape
    gblk = 64
    grid = (gb // gblk,)

    def body(seq_ref, idx_ref, tgt_ref, wq_ref, wk_ref, wp_ref, b1_ref,
             a1_ref, w2_ref, b2_ref, a2_ref, w3_ref, e4_ref, out_ref):
        seq = seq_ref[...]
        k2 = seq.reshape(ll * gblk, dp)
        qp = tgt_ref[...]
        qb = jnp.concatenate([qp] * ll, axis=0)
        qw = qp @ wq_ref[...] + b1_ref[...]
        pre1 = (
            k2 @ wk_ref[...]
            + (qb * k2) @ wp_ref[...]
            + jnp.concatenate([qw] * ll, axis=0)
        )
        h1 = jnp.where(pre1 > 0, pre1, a1_ref[...] * pre1)
        pre2 = h1 @ w2_ref[...] + b2_ref[...]
        h2 = jnp.where(pre2 > 0, pre2, a2_ref[...] * pre2)
        sc2 = h2 @ w3_ref[...]
        sc3 = sc2.reshape(ll, gblk, pack)
        mask = idx_ref[...] != 0
        scores = jnp.where(mask, sc3, jnp.float32(-1e9))
        m = jnp.max(scores, axis=0, keepdims=True)
        e = jnp.exp(scores - m)
        attn = e / jnp.sum(e, axis=0, keepdims=True)
        attnp = (attn.reshape(ll * gblk, pack) @ e4_ref[...])
        out_ref[...] = jnp.sum(attnp.reshape(ll, gblk, dp) * seq, axis=0)

    full = lambda shape: pl.BlockSpec(shape, lambda i: tuple(0 for _ in shape))
    return pl.pallas_call(
        body,
        grid=grid,
        in_specs=[
            pl.BlockSpec((ll, gblk, dp), lambda i: (0, i, 0)),
            pl.BlockSpec((ll, gblk, pack), lambda i: (0, i, 0)),
            pl.BlockSpec((gblk, dp), lambda i: (i, 0)),
            full(wqbd.shape), full(wkbd.shape), full(wpbd.shape),
            full(b1t.shape), full(a1t.shape), full(w2bd.shape),
            full(b2t.shape), full(a2t.shape), full(w3bd.shape),
            full(e4.shape),
        ],
        out_specs=pl.BlockSpec((gblk, dp), lambda i: (i, 0)),
        out_shape=jax.ShapeDtypeStruct((gb, dp), jnp.float32),
    )(seqp, idxp, tgtp, wqbd, wkbd, wpbd, b1t, a1t, w2bd, b2t, a2t, w3bd, e4)


def kernel(dense_inputs, sparse_inputs, seq_inputs, item_inputs, table,
           W1, b1, a1, W2, b2, a2, W3, b3):
    b, l, _ = seq_inputs.shape
    d = table.shape[1]
    idx_t = seq_inputs[:, :, 0].astype(jnp.int32).T
    item_idx = item_inputs[:, 0].astype(jnp.int32)

    v = table.shape[0]
    table_rm = _sc_transpose(table.T).reshape(v, d)
    seq_rows, tgt_rows = _sc_gather(table_rm, idx_t.reshape(l * b), item_idx)

    pack = 128 // d
    seqp = seq_rows.reshape(l, b // pack, pack * d)
    idxp = idx_t.reshape(l, b // pack, pack)
    tgtp = tgt_rows.reshape(b // pack, pack * d)

    w1q, w1k, w1d, w1p = W1[:d], W1[d:2 * d], W1[2 * d:3 * d], W1[3 * d:]
    eye = jnp.eye(pack, dtype=jnp.float32)
    bd = lambda w: jnp.kron(eye, w)
    tile = lambda x: jnp.tile(x, pack).reshape(1, -1)
    user_info = _tc_din(
        seqp, idxp, tgtp,
        bd(w1q + w1d), bd(w1k - w1d), bd(w1p),
        tile(b1), tile(a1),
        bd(W2), tile(b2), tile(a2),
        bd(W3),
        jnp.kron(eye, jnp.ones((1, d), jnp.float32)),
        pack,
    )
    return user_info.reshape(b, d)


# speedup vs baseline: 4.4913x; 1.1282x over previous
import functools

import jax
import jax.numpy as jnp
from jax import lax
from jax.experimental import pallas as pl
from jax.experimental.pallas import tpu as pltpu
from jax.experimental.pallas import tpu_sc as plsc


def _sc_transpose(table_t):
    d, v = table_t.shape
    pitch = d + 1
    lanes = 512
    n_stripes = v // lanes
    tail = v - n_stripes * lanes
    info = plsc.get_sparse_core_info()
    nw = info.num_cores * info.num_subcores
    main_steps = n_stripes // nw
    rem_stripes = n_stripes - main_steps * nw
    elems = lanes * d
    n_chunks = lanes // 16

    mesh = plsc.VectorSubcoreMesh(core_axis_name="c", subcore_axis_name="s")

    @functools.partial(
        pl.kernel,
        mesh=mesh,
        compiler_params=pltpu.CompilerParams(use_tc_tiling_on_sc=True,
                                             needs_layout_passes=False),
        out_type=jax.ShapeDtypeStruct((v * d,), jnp.float32),
        scratch_types=[
            pltpu.VMEM((d, lanes), jnp.float32),
            pltpu.VMEM((d, lanes), jnp.float32),
            pltpu.VMEM((lanes * pitch,), jnp.float32),
            pltpu.VMEM((elems,), jnp.float32),
            pltpu.VMEM((elems,), jnp.float32),
            pltpu.VMEM((d, tail), jnp.float32) if tail else None,
            pltpu.SemaphoreType.DMA,
            pltpu.SemaphoreType.DMA,
            pltpu.SemaphoreType.DMA,
            pltpu.SemaphoreType.DMA,
        ],
    )
    def transpose_k(tt_hbm, out_hbm, colbuf0, colbuf1, stage, rowbuf0,
                    rowbuf1, tailbuf, sem_in0, sem_in1, sem_out0, sem_out1):
        colbuf = (colbuf0, colbuf1)
        rowbuf = (rowbuf0, rowbuf1)
        sem_in = (sem_in0, sem_in1)
        sem_out = (sem_out0, sem_out1)
        wid = lax.axis_index("s") * info.num_cores + lax.axis_index("c")
        iota = jnp.arange(16, dtype=jnp.int32)
        iotap = iota * pitch

        def issue_in(t, b):
            s = wid + t * nw
            return pltpu.make_async_copy(
                tt_hbm.at[:, pl.ds(s * lanes, lanes)], colbuf[b], sem_in[b])

        def issue_out(t, b):
            s = wid + t * nw
            return pltpu.make_async_copy(
                rowbuf[b], out_hbm.at[pl.ds(s * elems, elems)], sem_out[b])

        def transpose_block(cb, rb, n_rows):
            def d_body(di, _):
                for k0 in range(0, n_rows // 16, 8):
                    nj = min(8, n_rows // 16 - k0)
                    vs = [cb[di, pl.ds((k0 + j) * 16, 16)] for j in range(nj)]
                    for j in range(nj):
                        plsc.store_scatter(
                            stage,
                            [iotap + ((k0 + j) * 16 * pitch + di)], vs[j])
                return 0
            lax.fori_loop(0, d, d_body, 0)

            def r_body(i, _):
                base = i * 4
                srcs = [(h, half) for h in range(4) for half in range(d // 16)]
                vs = [plsc.load_gather(
                    stage, [iota + ((base + h) * pitch + half * 16)])
                    for (h, half) in srcs]
                for (h, half), vv in zip(srcs, vs):
                    rb[pl.ds((base + h) * d + half * 16, 16)] = vv
                return 0
            lax.fori_loop(0, n_rows // 4, r_body, 0)

        issue_in(0, 0).start()

        def step(t, b):
            pltpu.make_async_copy(
                tt_hbm.at[:, pl.ds(0, lanes)], colbuf[b], sem_in[b]).wait()

            @pl.when(t + 1 < main_steps)
            def _():
                issue_in(t + 1, 1 - b).start()

            @pl.when(t >= 2)
            def _():
                pltpu.make_async_copy(
                    rowbuf[b], out_hbm.at[pl.ds(0, elems)], sem_out[b]).wait()

            transpose_block(colbuf[b], rowbuf[b], lanes)
            issue_out(t, b).start()

        def pair_body(u, _):
            step(2 * u, 0)
            step(2 * u + 1, 1)
            return 0

        lax.fori_loop(0, main_steps // 2, pair_body, 0)
        if main_steps % 2:
            step(main_steps - 1, 0)

        for b in range(2):
            if main_steps > 1 - b:
                pltpu.make_async_copy(
                    rowbuf[b], out_hbm.at[pl.ds(0, elems)], sem_out[b]).wait()

        for r in range(rem_stripes):
            @pl.when(wid == r)
            def _():
                s = main_steps * nw + r
                pltpu.sync_copy(tt_hbm.at[:, pl.ds(s * lanes, lanes)],
                                colbuf[0])
                transpose_block(colbuf[0], rowbuf[0], lanes)
                pltpu.sync_copy(rowbuf[0],
                                out_hbm.at[pl.ds(s * elems, elems)])

        if tail:
            @pl.when(wid == rem_stripes)
            def _():
                base = n_stripes * lanes
                pltpu.sync_copy(tt_hbm.at[:, pl.ds(base, tail)], tailbuf)
                transpose_block(tailbuf, rowbuf[0], tail)
                pltpu.sync_copy(rowbuf[0].at[pl.ds(0, tail * d)],
                                out_hbm.at[pl.ds(base * d, tail * d)])

    return transpose_k(table_t)


def _sc_gather(table, seq_idx_t, item_idx):
    info = plsc.get_sparse_core_info()
    nw = info.num_cores * info.num_subcores
    n_seq = seq_idx_t.shape[0]
    n_tgt = item_idx.shape[0]
    d = table.shape[1]
    seq_pw = n_seq // nw
    tgt_pw = n_tgt // nw
    ch = 1600
    n_ch = seq_pw // ch

    mesh = plsc.VectorSubcoreMesh(core_axis_name="c", subcore_axis_name="s")

    @functools.partial(
        pl.kernel,
        mesh=mesh,
        compiler_params=pltpu.CompilerParams(use_tc_tiling_on_sc=False),
        out_type=(
            jax.ShapeDtypeStruct((n_seq, d), jnp.float32),
            jax.ShapeDtypeStruct((n_tgt, d), jnp.float32),
        ),
        scratch_types=[
            pltpu.VMEM((ch,), jnp.int32),
            pltpu.VMEM((ch, d), jnp.float32),
            pltpu.VMEM((tgt_pw,), jnp.int32),
            pltpu.VMEM((tgt_pw, d), jnp.float32),
            pltpu.SemaphoreType.DMA,
        ],
    )
    def gather_k(table_hbm, seq_idx_hbm, item_idx_hbm, out_seq_hbm,
                 out_tgt_hbm, idx_v, rows_v, tidx_v, trows_v, sem):
        wid = lax.axis_index("s") * info.num_cores + lax.axis_index("c")
        tbase = wid * tgt_pw
        pltpu.sync_copy(item_idx_hbm.at[pl.ds(tbase, tgt_pw)], tidx_v)
        pltpu.async_copy(table_hbm.at[tidx_v], trows_v, sem).wait()
        pltpu.sync_copy(trows_v, out_tgt_hbm.at[pl.ds(tbase, tgt_pw)])
        sbase = wid * seq_pw
        for c in range(n_ch):
            off = sbase + c * ch
            pltpu.sync_copy(seq_idx_hbm.at[pl.ds(off, ch)], idx_v)
            pltpu.async_copy(table_hbm.at[idx_v], rows_v, sem).wait()
            pltpu.sync_copy(rows_v, out_seq_hbm.at[pl.ds(off, ch)])

    return gather_k(table, seq_idx_t, item_idx)


def _tc_din(seqp, idxp, tgtp, wqbd, wkbd, wpbd, b1t, a1t, w2bd, b2t, a2t,
            w3bd, e4, pack):
    ll, gb, dp = seqp.shape
    gblk = 64
    grid = (gb // gblk,)

    def body(seq_ref, idx_ref, tgt_ref, wq_ref, wk_ref, wp_ref, b1_ref,
             a1_ref, w2_ref, b2_ref, a2_ref, w3_ref, e4_ref, out_ref):
        seq = seq_ref[...]
        k2 = seq.reshape(ll * gblk, dp)
        qp = tgt_ref[...]
        qb = jnp.concatenate([qp] * ll, axis=0)
        qw = qp @ wq_ref[...] + b1_ref[...]
        pre1 = (
            k2 @ wk_ref[...]
            + (qb * k2) @ wp_ref[...]
            + jnp.concatenate([qw] * ll, axis=0)
        )
        h1 = jnp.where(pre1 > 0, pre1, a1_ref[...] * pre1)
        pre2 = h1 @ w2_ref[...] + b2_ref[...]
        h2 = jnp.where(pre2 > 0, pre2, a2_ref[...] * pre2)
        sc2 = h2 @ w3_ref[...]
        sc3 = sc2.reshape(ll, gblk, pack)
        mask = idx_ref[...] != 0
        scores = jnp.where(mask, sc3, jnp.float32(-1e9))
        m = jnp.max(scores, axis=0, keepdims=True)
        e = jnp.exp(scores - m)
        attn = e / jnp.sum(e, axis=0, keepdims=True)
        attnp = (attn.reshape(ll * gblk, pack) @ e4_ref[...])
        out_ref[...] = jnp.sum(attnp.reshape(ll, gblk, dp) * seq, axis=0)

    full = lambda shape: pl.BlockSpec(shape, lambda i: tuple(0 for _ in shape))
    return pl.pallas_call(
        body,
        grid=grid,
        in_specs=[
            pl.BlockSpec((ll, gblk, dp), lambda i: (0, i, 0)),
            pl.BlockSpec((ll, gblk, pack), lambda i: (0, i, 0)),
            pl.BlockSpec((gblk, dp), lambda i: (i, 0)),
            full(wqbd.shape), full(wkbd.shape), full(wpbd.shape),
            full(b1t.shape), full(a1t.shape), full(w2bd.shape),
            full(b2t.shape), full(a2t.shape), full(w3bd.shape),
            full(e4.shape),
        ],
        out_specs=pl.BlockSpec((gblk, dp), lambda i: (i, 0)),
        out_shape=jax.ShapeDtypeStruct((gb, dp), jnp.float32),
    )(seqp, idxp, tgtp, wqbd, wkbd, wpbd, b1t, a1t, w2bd, b2t, a2t, w3bd, e4)


def kernel(dense_inputs, sparse_inputs, seq_inputs, item_inputs, table,
           W1, b1, a1, W2, b2, a2, W3, b3):
    b, l, _ = seq_inputs.shape
    d = table.shape[1]
    idx_t = seq_inputs[:, :, 0].astype(jnp.int32).T
    item_idx = item_inputs[:, 0].astype(jnp.int32)

    v = table.shape[0]
    table_rm = _sc_transpose(table.T).reshape(v, d)
    seq_rows, tgt_rows = _sc_gather(table_rm, idx_t.reshape(l * b), item_idx)

    pack = 128 // d
    seqp = seq_rows.reshape(l, b // pack, pack * d)
    idxp = idx_t.reshape(l, b // pack, pack)
    tgtp = tgt_rows.reshape(b // pack, pack * d)

    w1q, w1k, w1d, w1p = W1[:d], W1[d:2 * d], W1[2 * d:3 * d], W1[3 * d:]
    eye = jnp.eye(pack, dtype=jnp.float32)
    bd = lambda w: jnp.kron(eye, w)
    tile = lambda x: jnp.tile(x, pack).reshape(1, -1)
    user_info = _tc_din(
        seqp, idxp, tgtp,
        bd(w1q + w1d), bd(w1k - w1d), bd(w1p),
        tile(b1), tile(a1),
        bd(W2), tile(b2), tile(a2),
        bd(W3),
        jnp.kron(eye, jnp.ones((1, d), jnp.float32)),
        pack,
    )
    return user_info.reshape(b, d)
